# Initial kernel scaffold; baseline (speedup 1.0000x reference)
import jax
import jax.numpy as jnp
from jax.experimental import pallas as pl


def kernel(features, edge_index, W1, b1, W2, b2):
    raise NotImplementedError("write your pallas kernel here")


# speedup vs baseline: 6.7254x; 6.7254x over previous
import functools

import jax
import jax.numpy as jnp
from jax import lax
from jax.experimental import pallas as pl
from jax.experimental.pallas import tpu as pltpu
from jax.experimental.pallas import tpu_sc as plsc

_NC = 2
_NS = 16
_IW = 128
_KB = 8


def _sc_hist(src2d, dst2d, vsrc, vdst, zeros):
  R = src2d.shape[0]
  NP = zeros.shape[0]
  n_tiles = _NC * _NS
  rt = R // n_tiles
  rpt = NP // _NS
  mesh = plsc.VectorSubcoreMesh(core_axis_name="c", subcore_axis_name="s")
  out = jax.ShapeDtypeStruct((_NC, NP, 128), jnp.float32)

  @functools.partial(
      pl.kernel, mesh=mesh, out_type=out,
      scratch_types=[
          pltpu.VMEM_SHARED((NP, 128), jnp.float32),
          pltpu.VMEM((_KB, _IW), jnp.int32),
          pltpu.VMEM((_KB, _IW), jnp.int32),
          pltpu.VMEM((_IW, 128), jnp.float32),
          pltpu.VMEM((_IW, 128), jnp.float32),
      ])
  def k(src_h, dst_h, vs_h, vd_h, zeros_h, o_h,
        acc, idx_s, idx_d, vs_v, vd_v):
    c = lax.axis_index("c")
    s = lax.axis_index("s")
    wid = c * _NS + s
    zb = s * rpt
    pltpu.sync_copy(zeros_h.at[pl.ds(zb, rpt)], acc.at[pl.ds(zb, rpt)])
    pltpu.sync_copy(vs_h, vs_v)
    pltpu.sync_copy(vd_h, vd_v)
    plsc.subcore_barrier()

    @pl.loop(0, rt // _KB)
    def _(t):
      base = wid * rt + t * _KB
      pltpu.sync_copy(src_h.at[pl.ds(base, _KB)], idx_s)
      pltpu.sync_copy(dst_h.at[pl.ds(base, _KB)], idx_d)
      for j in range(_KB):
        pltpu.sync_copy(vs_v, acc.at[idx_s.at[j]], add=True)
        pltpu.sync_copy(vd_v, acc.at[idx_d.at[j]], add=True)

    plsc.subcore_barrier()
    pltpu.sync_copy(acc.at[pl.ds(zb, rpt)], o_h.at[c, pl.ds(zb, rpt)])

  return k(src2d, dst2d, vsrc, vdst, zeros)


def _sc_spmm(table, src2d, dst2d, zeros):
  NP, D = table.shape
  R = src2d.shape[0]
  n_tiles = _NC * _NS
  rt = R // n_tiles
  rpt = NP // _NS
  mesh = plsc.VectorSubcoreMesh(core_axis_name="c", subcore_axis_name="s")
  out = jax.ShapeDtypeStruct((_NC, NP, D), jnp.float32)

  @functools.partial(
      pl.kernel, mesh=mesh, out_type=out,
      scratch_types=[
          pltpu.VMEM_SHARED((NP, D), jnp.float32),
          pltpu.VMEM((_KB, _IW), jnp.int32),
          pltpu.VMEM((_KB, _IW), jnp.int32),
          pltpu.VMEM((_IW, D), jnp.float32),
      ])
  def k(tab_h, src_h, dst_h, zeros_h, o_h, acc, idx_s, idx_d, rows_v):
    c = lax.axis_index("c")
    s = lax.axis_index("s")
    wid = c * _NS + s
    zb = s * rpt
    pltpu.sync_copy(zeros_h.at[pl.ds(zb, rpt)], acc.at[pl.ds(zb, rpt)])
    plsc.subcore_barrier()

    @pl.loop(0, rt // _KB)
    def _(t):
      base = wid * rt + t * _KB
      pltpu.sync_copy(src_h.at[pl.ds(base, _KB)], idx_s)
      pltpu.sync_copy(dst_h.at[pl.ds(base, _KB)], idx_d)
      for j in range(_KB):
        pltpu.sync_copy(tab_h.at[idx_s.at[j]], rows_v)
        pltpu.sync_copy(rows_v, acc.at[idx_d.at[j]], add=True)

    plsc.subcore_barrier()
    pltpu.sync_copy(acc.at[pl.ds(zb, rpt)], o_h.at[c, pl.ds(zb, rpt)])

  return k(table, src2d, dst2d, zeros)


_BT = 1264


def _tc_matmul(x, w):
  n, din = x.shape
  dout = w.shape[1]

  def body(x_ref, w_ref, o_ref):
    o_ref[...] = jnp.dot(x_ref[...], w_ref[...],
                         preferred_element_type=jnp.float32,
                         precision=lax.Precision.HIGHEST)

  return pl.pallas_call(
      body,
      grid=(n // _BT,),
      in_specs=[pl.BlockSpec((_BT, din), lambda i: (i, 0)),
                pl.BlockSpec((din, dout), lambda i: (0, 0))],
      out_specs=pl.BlockSpec((_BT, dout), lambda i: (i, 0)),
      out_shape=jax.ShapeDtypeStruct((n, dout), jnp.float32),
  )(x, w)


def _tc_norms(hist, z):
  n, d = z.shape

  def body(h_ref, z_ref, h1_ref, ns_ref, nd_ref):
    deg = h_ref[0] + h_ref[1]
    deg_out = deg[:, 0:1]
    deg_in = deg[:, 64:65]
    ns = jnp.where(deg_out > 0, lax.rsqrt(jnp.maximum(deg_out, 1.0)), 0.0)
    nd = jnp.where(deg_in > 0, lax.rsqrt(jnp.maximum(deg_in, 1.0)), 0.0)
    ns_ref[...] = ns
    nd_ref[...] = nd
    h1_ref[...] = z_ref[...] * ns

  return pl.pallas_call(
      body,
      grid=(n // _BT,),
      in_specs=[pl.BlockSpec((_NC, _BT, 128), lambda i: (0, i, 0)),
                pl.BlockSpec((_BT, d), lambda i: (i, 0))],
      out_specs=[pl.BlockSpec((_BT, d), lambda i: (i, 0)),
                 pl.BlockSpec((_BT, 1), lambda i: (i, 0)),
                 pl.BlockSpec((_BT, 1), lambda i: (i, 0))],
      out_shape=[jax.ShapeDtypeStruct((n, d), jnp.float32),
                 jax.ShapeDtypeStruct((n, 1), jnp.float32),
                 jax.ShapeDtypeStruct((n, 1), jnp.float32)],
  )(hist, z)


def _tc_layer(p, nd, ns, b1):
  n = p.shape[1]
  dh = p.shape[2]

  def body(p_ref, nd_ref, ns_ref, b1_ref, o_ref):
    a = (p_ref[0] + p_ref[1]) * nd_ref[...]
    x1 = jnp.maximum(a + b1_ref[...], 0.0)
    o_ref[...] = x1 * ns_ref[...]

  return pl.pallas_call(
      body,
      grid=(n // _BT,),
      in_specs=[pl.BlockSpec((_NC, _BT, dh), lambda i: (0, i, 0)),
                pl.BlockSpec((_BT, 1), lambda i: (i, 0)),
                pl.BlockSpec((_BT, 1), lambda i: (i, 0)),
                pl.BlockSpec((1, dh), lambda i: (0, 0))],
      out_specs=pl.BlockSpec((_BT, dh), lambda i: (i, 0)),
      out_shape=jax.ShapeDtypeStruct((n, dh), jnp.float32),
  )(p, nd, ns, b1)


def _tc_final(q, nd, w2, b2):
  n = q.shape[1]
  dh = q.shape[2]
  do = w2.shape[1]

  def body(q_ref, nd_ref, w2_ref, b2_ref, o_ref):
    agg = (q_ref[0] + q_ref[1]) * nd_ref[...]
    o_ref[...] = jnp.dot(agg, w2_ref[...],
                         preferred_element_type=jnp.float32,
                         precision=lax.Precision.HIGHEST) + b2_ref[...]

  return pl.pallas_call(
      body,
      grid=(n // _BT,),
      in_specs=[pl.BlockSpec((_NC, _BT, dh), lambda i: (0, i, 0)),
                pl.BlockSpec((_BT, 1), lambda i: (i, 0)),
                pl.BlockSpec((dh, do), lambda i: (0, 0)),
                pl.BlockSpec((1, do), lambda i: (0, 0))],
      out_specs=pl.BlockSpec((_BT, do), lambda i: (i, 0)),
      out_shape=jax.ShapeDtypeStruct((n, do), jnp.float32),
  )(q, nd, w2, b2)


def kernel(features, edge_index, W1, b1, W2, b2):
  n, din = features.shape
  e = edge_index.shape[1]

  np_ = ((n + _NS * 8 - 1) // (_NS * 8)) * (_NS * 8)
  slab = _IW * _NC * _NS * _KB
  ep = ((e + slab - 1) // slab) * slab
  pad_n = np_ - n
  pad_e = ep - e

  xp = jnp.concatenate(
      [features, jnp.zeros((pad_n, din), jnp.float32)], axis=0)
  pad_idx = n + (jnp.arange(pad_e, dtype=jnp.int32) % pad_n)
  src2d = jnp.concatenate([edge_index[0], pad_idx]).reshape(-1, _IW)
  dst2d = jnp.concatenate([edge_index[1], pad_idx]).reshape(-1, _IW)
  lanes = jnp.arange(128)
  vsrc = jnp.broadcast_to((lanes == 0).astype(jnp.float32), (_IW, 128))
  vdst = jnp.broadcast_to((lanes == 64).astype(jnp.float32), (_IW, 128))
  zeros = jnp.zeros((np_, 128), jnp.float32)

  hist = _sc_hist(src2d, dst2d, vsrc, vdst, zeros)
  z = _tc_matmul(xp, W1)
  h1, ns, nd = _tc_norms(hist, z)
  p = _sc_spmm(h1, src2d, dst2d, zeros)
  t2 = _tc_layer(p, nd, ns, b1.reshape(1, -1))
  q = _sc_spmm(t2, src2d, dst2d, zeros)
  return _tc_final(q, nd, W2, b2.reshape(1, -1))[:n]


# speedup vs baseline: 10.3653x; 1.5412x over previous
import functools

import jax
import jax.numpy as jnp
from jax import lax
from jax.experimental import pallas as pl
from jax.experimental.pallas import tpu as pltpu
from jax.experimental.pallas import tpu_sc as plsc

_NC = 2
_NS = 16
_IW = 128
_KB = 8


def _sc_hist(src2d, dst2d, ones, zeros):
  R = src2d.shape[0]
  NP = zeros.shape[0]
  n_tiles = _NC * _NS
  rt = R // n_tiles
  mesh = plsc.VectorSubcoreMesh(core_axis_name="c", subcore_axis_name="s")
  out = jax.ShapeDtypeStruct((_NC, NP), jnp.float32)

  @functools.partial(
      pl.kernel, mesh=mesh, out_type=(out, out),
      scratch_types=[
          pltpu.VMEM_SHARED((NP,), jnp.float32),
          pltpu.VMEM_SHARED((NP,), jnp.float32),
          pltpu.VMEM((_KB, _IW), jnp.int32),
          pltpu.VMEM((_KB, _IW), jnp.int32),
          pltpu.VMEM((_IW,), jnp.float32),
      ])
  def k(src_h, dst_h, ones_h, zeros_h, os_h, od_h,
        acc_s, acc_d, idx_s, idx_d, ones_v):
    c = lax.axis_index("c")
    s = lax.axis_index("s")
    wid = c * _NS + s
    @pl.when(s == 0)
    def _():
      pltpu.sync_copy(zeros_h, acc_s)
      pltpu.sync_copy(zeros_h, acc_d)
    pltpu.sync_copy(ones_h, ones_v)
    plsc.subcore_barrier()

    @pl.loop(0, rt // _KB)
    def _(t):
      base = wid * rt + t * _KB
      pltpu.sync_copy(src_h.at[pl.ds(base, _KB)], idx_s)
      pltpu.sync_copy(dst_h.at[pl.ds(base, _KB)], idx_d)
      for j in range(_KB):
        pltpu.sync_copy(ones_v, acc_s.at[idx_s.at[j]], add=True)
        pltpu.sync_copy(ones_v, acc_d.at[idx_d.at[j]], add=True)

    plsc.subcore_barrier()
    @pl.when(s == 0)
    def _():
      pltpu.sync_copy(acc_s, os_h.at[c])
      pltpu.sync_copy(acc_d, od_h.at[c])

  return k(src2d, dst2d, ones, zeros)


def _sc_spmm(table, src2d, dst2d, zeros):
  NP, D = table.shape
  R = src2d.shape[0]
  n_tiles = _NC * _NS
  rt = R // n_tiles
  rpt = NP // _NS
  mesh = plsc.VectorSubcoreMesh(core_axis_name="c", subcore_axis_name="s")
  out = jax.ShapeDtypeStruct((_NC, NP, D), jnp.float32)

  nt = rt // _KB

  @functools.partial(
      pl.kernel, mesh=mesh, out_type=out,
      scratch_types=[
          pltpu.VMEM_SHARED((NP, D), jnp.float32),
          pltpu.VMEM((2 * _KB, _IW), jnp.int32),
          pltpu.VMEM((2 * _KB, _IW), jnp.int32),
          pltpu.VMEM((_IW, D), jnp.float32),
          pltpu.VMEM((_IW, D), jnp.float32),
          pltpu.SemaphoreType.DMA,
          pltpu.SemaphoreType.DMA,
      ])
  def k(tab_h, src_h, dst_h, zeros_h, o_h,
        acc, idx_s, idx_d, rows0, rows1, gsem, isem):
    c = lax.axis_index("c")
    s = lax.axis_index("s")
    wid = c * _NS + s
    zb = s * rpt
    pltpu.sync_copy(zeros_h.at[pl.ds(zb, rpt)], acc.at[pl.ds(zb, rpt)])
    plsc.subcore_barrier()

    base0 = wid * rt
    pltpu.sync_copy(src_h.at[pl.ds(base0, _KB)], idx_s.at[pl.ds(0, _KB)])
    pltpu.sync_copy(dst_h.at[pl.ds(base0, _KB)], idx_d.at[pl.ds(0, _KB)])
    pltpu.make_async_copy(tab_h.at[idx_s.at[0]], rows0, gsem).start()

    @pl.loop(0, nt)
    def _(t):
      par = lax.rem(t, 2)
      pb = par * _KB
      nb = (1 - par) * _KB
      nbase = wid * rt + (t + 1) * _KB
      not_last = t < (nt - 1)

      @pl.when(not_last)
      def _():
        pltpu.make_async_copy(src_h.at[pl.ds(nbase, _KB)],
                              idx_s.at[pl.ds(nb, _KB)], isem).start()
        pltpu.make_async_copy(dst_h.at[pl.ds(nbase, _KB)],
                              idx_d.at[pl.ds(nb, _KB)], isem).start()

      for j in range(_KB):
        rbuf = rows0 if j % 2 == 0 else rows1
        nbuf = rows1 if j % 2 == 0 else rows0
        pltpu.make_async_copy(tab_h.at[idx_s.at[pb + j]], rbuf, gsem).wait()
        if j + 1 < _KB:
          pltpu.make_async_copy(
              tab_h.at[idx_s.at[pb + j + 1]], nbuf, gsem).start()
        else:
          @pl.when(not_last)
          def _():
            pltpu.make_async_copy(src_h.at[pl.ds(nbase, _KB)],
                                  idx_s.at[pl.ds(nb, _KB)], isem).wait()
            pltpu.make_async_copy(dst_h.at[pl.ds(nbase, _KB)],
                                  idx_d.at[pl.ds(nb, _KB)], isem).wait()
            pltpu.make_async_copy(tab_h.at[idx_s.at[nb]], nbuf, gsem).start()
        pltpu.sync_copy(rbuf, acc.at[idx_d.at[pb + j]], add=True)

    plsc.subcore_barrier()
    pltpu.sync_copy(acc.at[pl.ds(zb, rpt)], o_h.at[c, pl.ds(zb, rpt)])

  return k(table, src2d, dst2d, zeros)


_BT = 1264


def _tc_matmul(x, w):
  n, din = x.shape
  dout = w.shape[1]

  def body(x_ref, w_ref, o_ref):
    o_ref[...] = jnp.dot(x_ref[...], w_ref[...],
                         preferred_element_type=jnp.float32,
                         precision=lax.Precision.HIGHEST)

  return pl.pallas_call(
      body,
      grid=(n // _BT,),
      in_specs=[pl.BlockSpec((_BT, din), lambda i: (i, 0)),
                pl.BlockSpec((din, dout), lambda i: (0, 0))],
      out_specs=pl.BlockSpec((_BT, dout), lambda i: (i, 0)),
      out_shape=jax.ShapeDtypeStruct((n, dout), jnp.float32),
  )(x, w)


def _tc_norms(hs, hd, z):
  n, d = z.shape

  def body(hs_ref, hd_ref, z_ref, h1_ref, ns_ref, nd_ref):
    deg_out = hs_ref[0] + hs_ref[1]
    deg_in = hd_ref[0] + hd_ref[1]
    ns = jnp.where(deg_out > 0, lax.rsqrt(jnp.maximum(deg_out, 1.0)), 0.0)
    nd = jnp.where(deg_in > 0, lax.rsqrt(jnp.maximum(deg_in, 1.0)), 0.0)
    ns_ref[...] = ns
    nd_ref[...] = nd
    h1_ref[...] = z_ref[...] * ns

  return pl.pallas_call(
      body,
      grid=(n // _BT,),
      in_specs=[pl.BlockSpec((_NC, _BT, 1), lambda i: (0, i, 0)),
                pl.BlockSpec((_NC, _BT, 1), lambda i: (0, i, 0)),
                pl.BlockSpec((_BT, d), lambda i: (i, 0))],
      out_specs=[pl.BlockSpec((_BT, d), lambda i: (i, 0)),
                 pl.BlockSpec((_BT, 1), lambda i: (i, 0)),
                 pl.BlockSpec((_BT, 1), lambda i: (i, 0))],
      out_shape=[jax.ShapeDtypeStruct((n, d), jnp.float32),
                 jax.ShapeDtypeStruct((n, 1), jnp.float32),
                 jax.ShapeDtypeStruct((n, 1), jnp.float32)],
  )(hs, hd, z)


def _tc_layer(p, nd, ns, b1):
  n = p.shape[1]
  dh = p.shape[2]

  def body(p_ref, nd_ref, ns_ref, b1_ref, o_ref):
    a = (p_ref[0] + p_ref[1]) * nd_ref[...]
    x1 = jnp.maximum(a + b1_ref[...], 0.0)
    o_ref[...] = x1 * ns_ref[...]

  return pl.pallas_call(
      body,
      grid=(n // _BT,),
      in_specs=[pl.BlockSpec((_NC, _BT, dh), lambda i: (0, i, 0)),
                pl.BlockSpec((_BT, 1), lambda i: (i, 0)),
                pl.BlockSpec((_BT, 1), lambda i: (i, 0)),
                pl.BlockSpec((1, dh), lambda i: (0, 0))],
      out_specs=pl.BlockSpec((_BT, dh), lambda i: (i, 0)),
      out_shape=jax.ShapeDtypeStruct((n, dh), jnp.float32),
  )(p, nd, ns, b1)


def _tc_final(q, nd, w2, b2):
  n = q.shape[1]
  dh = q.shape[2]
  do = w2.shape[1]

  def body(q_ref, nd_ref, w2_ref, b2_ref, o_ref):
    agg = (q_ref[0] + q_ref[1]) * nd_ref[...]
    o_ref[...] = jnp.dot(agg, w2_ref[...],
                         preferred_element_type=jnp.float32,
                         precision=lax.Precision.HIGHEST) + b2_ref[...]

  return pl.pallas_call(
      body,
      grid=(n // _BT,),
      in_specs=[pl.BlockSpec((_NC, _BT, dh), lambda i: (0, i, 0)),
                pl.BlockSpec((_BT, 1), lambda i: (i, 0)),
                pl.BlockSpec((dh, do), lambda i: (0, 0)),
                pl.BlockSpec((1, do), lambda i: (0, 0))],
      out_specs=pl.BlockSpec((_BT, do), lambda i: (i, 0)),
      out_shape=jax.ShapeDtypeStruct((n, do), jnp.float32),
  )(q, nd, w2, b2)


def kernel(features, edge_index, W1, b1, W2, b2):
  n, din = features.shape
  e = edge_index.shape[1]

  np_ = ((n + _NS * 8 - 1) // (_NS * 8)) * (_NS * 8)
  slab = _IW * _NC * _NS * _KB
  ep = ((e + slab - 1) // slab) * slab
  pad_n = np_ - n
  pad_e = ep - e

  xp = jnp.concatenate(
      [features, jnp.zeros((pad_n, din), jnp.float32)], axis=0)
  pad_idx = n + (jnp.arange(pad_e, dtype=jnp.int32) % pad_n)
  src2d = jnp.concatenate([edge_index[0], pad_idx]).reshape(-1, _IW)
  dst2d = jnp.concatenate([edge_index[1], pad_idx]).reshape(-1, _IW)
  ones1 = jnp.ones((_IW,), jnp.float32)
  zeros1 = jnp.zeros((np_,), jnp.float32)
  zeros = jnp.zeros((np_, 128), jnp.float32)

  hs, hd = _sc_hist(src2d, dst2d, ones1, zeros1)
  z = _tc_matmul(xp, W1)
  h1, ns, nd = _tc_norms(hs.reshape(_NC, np_, 1), hd.reshape(_NC, np_, 1), z)
  p = _sc_spmm(h1, src2d, dst2d, zeros)
  t2 = _tc_layer(p, nd, ns, b1.reshape(1, -1))
  q = _sc_spmm(t2, src2d, dst2d, zeros)
  return _tc_final(q, nd, W2, b2.reshape(1, -1))[:n]
